# baseline (speedup 1.0000x reference)
import functools

import jax
import jax.numpy as jnp
from jax import lax
from jax.experimental import pallas as pl
from jax.experimental.pallas import tpu as pltpu
from jax.experimental.pallas import tpu_sc as plsc

_B = 16384
_D = 32
_NC = 2
_NS = 16
_NW = _NC * _NS
_BPW = _B // _NW

def _gather_body(user_tab, item_tab, uidx, iidx, uout, iout,
                 uidx_v, iidx_v, urows_v, irows_v, sem_u, sem_i):
    wid = lax.axis_index("s") * _NC + lax.axis_index("c")
    base = wid * _BPW
    pltpu.sync_copy(uidx.at[pl.ds(base, _BPW)], uidx_v)
    pltpu.sync_copy(iidx.at[pl.ds(base, _BPW)], iidx_v)
    cu = pltpu.async_copy(user_tab.at[uidx_v], urows_v, sem_u)
    ci = pltpu.async_copy(item_tab.at[iidx_v], irows_v, sem_i)
    cu.wait()
    ci.wait()
    pltpu.sync_copy(urows_v, uout.at[pl.ds(base, _BPW)])
    pltpu.sync_copy(irows_v, iout.at[pl.ds(base, _BPW)])


@functools.lru_cache(maxsize=None)
def _build_gather2():
    mesh = plsc.VectorSubcoreMesh(
        core_axis_name="c", subcore_axis_name="s",
        num_cores=_NC, num_subcores=_NS,
    )
    return pl.kernel(
        _gather_body,
        out_type=(
            jax.ShapeDtypeStruct((_B, _D), jnp.float32),
            jax.ShapeDtypeStruct((_B, _D), jnp.float32),
        ),
        mesh=mesh,
        compiler_params=pltpu.CompilerParams(use_tc_tiling_on_sc=False),
        scratch_types=[
            pltpu.VMEM((_BPW,), jnp.int32),
            pltpu.VMEM((_BPW,), jnp.int32),
            pltpu.VMEM((_BPW, _D), jnp.float32),
            pltpu.VMEM((_BPW, _D), jnp.float32),
            pltpu.SemaphoreType.DMA,
            pltpu.SemaphoreType.DMA,
        ],
    )


_BS = 2048


def _mlp_body(desc_ref, u_ref, i_ref, wd_ref, bd_ref, w1u_ref, w1i_ref,
              w1d_ref, b1_ref, w2_ref, b2_ref, wo_ref, bo_ref, out_ref):
    d = jnp.dot(desc_ref[...], wd_ref[...], preferred_element_type=jnp.float32)
    d = jnp.maximum(d + bd_ref[...], 0.0)
    h1 = jnp.dot(u_ref[...], w1u_ref[...], preferred_element_type=jnp.float32)
    h1 = h1 + jnp.dot(i_ref[...], w1i_ref[...], preferred_element_type=jnp.float32)
    h1 = h1 + jnp.dot(d, w1d_ref[...], preferred_element_type=jnp.float32)
    h1 = jnp.maximum(h1 + b1_ref[...], 0.0)
    h2 = jnp.dot(h1, w2_ref[...], preferred_element_type=jnp.float32)
    h2 = jnp.maximum(h2 + b2_ref[...], 0.0)
    out_ref[...] = jnp.dot(h2, wo_ref[...], preferred_element_type=jnp.float32) + bo_ref[...]


def _mlp(desc, u_emb, i_emb, wd, bd, w1u, w1i, w1d, b1, w2, b2, wo, bo):
    grid = (_B // _BS,)
    full = lambda shape: pl.BlockSpec(shape, lambda i: (0, 0))
    return pl.pallas_call(
        _mlp_body,
        grid=grid,
        in_specs=[
            pl.BlockSpec((_BS, 300), lambda i: (i, 0)),
            pl.BlockSpec((_BS, _D), lambda i: (i, 0)),
            pl.BlockSpec((_BS, _D), lambda i: (i, 0)),
            full((300, _D)),
            full((1, _D)),
            full((_D, 64)),
            full((_D, 64)),
            full((_D, 64)),
            full((1, 64)),
            full((64, 32)),
            full((1, 32)),
            full((32, 1)),
            full((1, 1)),
        ],
        out_specs=pl.BlockSpec((_BS, 1), lambda i: (i, 0)),
        out_shape=jax.ShapeDtypeStruct((_B, 1), jnp.float32),
    )(desc, u_emb, i_emb, wd, bd, w1u, w1i, w1d, b1, w2, b2, wo, bo)


def kernel(user_input, item_input, description_input, user_table, item_table,
           W_desc, b_desc, W1, b1, W2, b2, W_out, b_out):
    uidx = user_input.reshape(-1)
    iidx = item_input.reshape(-1)
    u_emb, i_emb = _build_gather2()(user_table, item_table, uidx, iidx)
    return _mlp(
        description_input, u_emb, i_emb,
        W_desc, b_desc.reshape(1, -1),
        W1[:_D], W1[_D:2 * _D], W1[2 * _D:], b1.reshape(1, -1),
        W2, b2.reshape(1, -1),
        W_out, b_out.reshape(1, -1),
    )
